# baseline (speedup 1.0000x reference)
import functools

import jax
import jax.numpy as jnp
from jax import lax
from jax.experimental import pallas as pl
from jax.experimental.pallas import tpu as pltpu
from jax.experimental.pallas import tpu_sc as plsc

_GROUP = 128
_K = 5
_MAX_A = 48
_TC_BLK = 1024


def _sc_gather(idx3, table, n_rows, dim):
    info = plsc.get_sparse_core_info()
    nc, ns = info.num_cores, info.num_subcores
    nw = nc * ns
    rpw = n_rows // nw
    ng = rpw // _GROUP
    sup = ng // _K
    cg = _K * _GROUP

    @functools.partial(
        pl.kernel,
        out_type=jax.ShapeDtypeStruct((n_rows, dim), jnp.float32),
        mesh=plsc.VectorSubcoreMesh(core_axis_name="c", subcore_axis_name="s"),
        scratch_types=[
            pltpu.VMEM((ng, _GROUP), jnp.int32),
            pltpu.VMEM((cg, dim), jnp.float32),
            pltpu.VMEM((cg, dim), jnp.float32),
            pltpu.SemaphoreType.DMA,
            pltpu.SemaphoreType.DMA,
        ],
        compiler_params=pltpu.CompilerParams(use_tc_tiling_on_sc=False),
    )
    def gather_k(idx_hbm, table_hbm, out_hbm, idx_v, buf_a, buf_b, sem_a, sem_b):
        c = lax.axis_index("c")
        s = lax.axis_index("s")
        wid = s * nc + c
        base = wid * rpw
        pltpu.sync_copy(idx_hbm.at[wid], idx_v)

        def issue(sc_i, buf, sem):
            for j in range(_K):
                pltpu.async_copy(
                    table_hbm.at[idx_v.at[sc_i * _K + j]],
                    buf.at[pl.ds(j * _GROUP, _GROUP)], sem)

        def drain(sc_i, buf, sem):
            for j in range(_K):
                pltpu.make_async_copy(
                    table_hbm.at[idx_v.at[sc_i * _K + j]],
                    buf.at[pl.ds(j * _GROUP, _GROUP)], sem).wait()

        def write(sc_i, buf):
            pltpu.sync_copy(buf, out_hbm.at[pl.ds(base + sc_i * cg, cg)])

        issue(0, buf_a, sem_a)

        def body(p, carry):
            sa = 2 * p
            sb = 2 * p + 1
            issue(sb, buf_b, sem_b)
            drain(sa, buf_a, sem_a)
            write(sa, buf_a)

            @pl.when(sb + 1 < sup)
            def _():
                issue(sb + 1, buf_a, sem_a)

            drain(sb, buf_b, sem_b)
            write(sb, buf_b)
            return carry

        lax.fori_loop(0, sup // 2, body, 0)

    return gather_k(idx3, table)


_SUB = 64

_EXP2_C = (1.0, 0.6931471805599453, 0.2402265069591007, 0.05550410866482158,
           0.009618129107628477, 0.0013333558146428443, 0.00015403530393381608)
_LOG1P_C = (6.057847667939598e-09, 0.9999987830867273, -0.49995894468480306,
            0.3327853380006574, -0.24618967719166315, 0.18421386356488162,
            -0.12447194563436599, 0.06573552558543269, -0.0226280072114605,
            0.003662242215796141)


def _fast_exp(y):
    z0 = y * jnp.float32(1.4426950408889634)
    z = jnp.maximum(z0, -124.5)
    n = jnp.round(z)
    f = z - n
    p = jnp.float32(_EXP2_C[-1])
    for c in _EXP2_C[-2::-1]:
        p = p * f + jnp.float32(c)
    scale = lax.bitcast_convert_type(
        (n.astype(jnp.int32) + 127) << 23, jnp.float32)
    return jnp.where(z0 < -124.5, 0.0, p * scale)


def _fast_log1p01(v):
    p = jnp.float32(_LOG1P_C[-1])
    for c in _LOG1P_C[-2::-1]:
        p = p * v + jnp.float32(c)
    return p


def _tc_body(k_ref, w_ref, o_ref):
    a_max = jnp.max(k_ref[...])
    w_max = jnp.max(w_ref[...])
    x_hi = jnp.maximum(w_max, 0.0) + 0.7
    lim = jnp.minimum(a_max, jnp.float32(_MAX_A))

    def conv_cond(c):
        j, t = c
        return jnp.logical_and(j < lim, t > 1e-8)

    def conv_step(c):
        j, t = c
        return (j + 1.0, t * (x_hi / (j + 1.0)))

    trip_f, _ = lax.while_loop(conv_cond, conv_step,
                               (jnp.float32(0.0), jnp.float32(1.0)))
    trip = trip_f.astype(jnp.int32)

    n_sub = _TC_BLK // _SUB
    for i in range(n_sub):
        rows = pl.ds(i * _SUB, _SUB)
        w = w_ref[rows, :]
        a = k_ref[rows, :]
        v = _fast_exp(-jnp.abs(w))
        x = jnp.maximum(w, 0.0) + _fast_log1p01(v)
        e = _fast_exp(-x)

        def step(j, carry):
            s, t = carry
            jf = j.astype(jnp.float32)
            s = s + jnp.where(a > jf, t, 0.0)
            t = t * (x * (1.0 / (jf + 1.0)))
            return (s, t)

        s, _ = lax.fori_loop(0, trip, step, (jnp.zeros_like(w), e))
        o_ref[rows, :] = 1.0 - s


def _tc_series(kin_exp, packed, p_rows):
    return pl.pallas_call(
        _tc_body,
        grid=(p_rows // _TC_BLK,),
        in_specs=[
            pl.BlockSpec((_TC_BLK, 128), lambda i: (i, 0)),
            pl.BlockSpec((_TC_BLK, 128), lambda i: (i, 0)),
        ],
        out_specs=pl.BlockSpec((_TC_BLK, 128), lambda i: (i, 0)),
        out_shape=jax.ShapeDtypeStruct((p_rows, 128), jnp.float32),
        compiler_params=pltpu.CompilerParams(
            dimension_semantics=("arbitrary",)),
    )(kin_exp, packed)


def kernel(problems, behavior_data, W):
    b, l = problems.shape
    dim = W.shape[1]
    n = b * l
    info = plsc.get_sparse_core_info()
    nw = info.num_cores * info.num_subcores

    idx3 = problems.reshape(nw, n // (nw * _GROUP), _GROUP)
    rows = _sc_gather(idx3, W, n, dim)

    p_rows = (n * dim) // 128
    packed = rows.reshape(p_rows, 128)
    a_flat = jnp.maximum(behavior_data.astype(jnp.float32) - 1.0, 0.0)
    kin_exp = jnp.repeat(a_flat.reshape(-1), dim).reshape(p_rows, 128)
    out = _tc_series(kin_exp, packed, p_rows)
    return out.reshape(b, l, dim)


# speedup vs baseline: 1.4267x; 1.4267x over previous
import functools

import jax
import jax.numpy as jnp
from jax import lax
from jax.experimental import pallas as pl
from jax.experimental.pallas import tpu as pltpu
from jax.experimental.pallas import tpu_sc as plsc

_GROUP = 128
_K = 5
_MAX_A = 48
_TC_BLK = 1024


def _sc_gather(idx3, table, n_rows, dim):
    info = plsc.get_sparse_core_info()
    nc, ns = info.num_cores, info.num_subcores
    nw = nc * ns
    rpw = n_rows // nw
    ng = rpw // _GROUP
    sup = ng // _K
    cg = _K * _GROUP

    @functools.partial(
        pl.kernel,
        out_type=jax.ShapeDtypeStruct((n_rows, dim), jnp.float32),
        mesh=plsc.VectorSubcoreMesh(core_axis_name="c", subcore_axis_name="s"),
        scratch_types=[
            pltpu.VMEM((ng, _GROUP), jnp.int32),
            pltpu.VMEM((cg, dim), jnp.float32),
            pltpu.VMEM((cg, dim), jnp.float32),
            pltpu.SemaphoreType.DMA,
            pltpu.SemaphoreType.DMA,
        ],
        compiler_params=pltpu.CompilerParams(use_tc_tiling_on_sc=False),
    )
    def gather_k(idx_hbm, table_hbm, out_hbm, idx_v, buf_a, buf_b, sem_a, sem_b):
        c = lax.axis_index("c")
        s = lax.axis_index("s")
        wid = s * nc + c
        base = wid * rpw
        pltpu.sync_copy(idx_hbm.at[wid], idx_v)

        def issue(sc_i, buf, sem):
            for j in range(_K):
                pltpu.async_copy(
                    table_hbm.at[idx_v.at[sc_i * _K + j]],
                    buf.at[pl.ds(j * _GROUP, _GROUP)], sem)

        def drain(sc_i, buf, sem):
            for j in range(_K):
                pltpu.make_async_copy(
                    table_hbm.at[idx_v.at[sc_i * _K + j]],
                    buf.at[pl.ds(j * _GROUP, _GROUP)], sem).wait()

        def write(sc_i, buf):
            pltpu.sync_copy(buf, out_hbm.at[pl.ds(base + sc_i * cg, cg)])

        issue(0, buf_a, sem_a)

        def body(p, carry):
            sa = 2 * p
            sb = 2 * p + 1
            issue(sb, buf_b, sem_b)
            drain(sa, buf_a, sem_a)
            write(sa, buf_a)

            @pl.when(sb + 1 < sup)
            def _():
                issue(sb + 1, buf_a, sem_a)

            drain(sb, buf_b, sem_b)
            write(sb, buf_b)
            return carry

        lax.fori_loop(0, sup // 2, body, 0)

    return gather_k(idx3, table)


_SUB = 64
_CHUNK = 8

_EXP2_C = (1.0, 0.6931471805599453, 0.2402265069591007, 0.05550410866482158,
           0.009618129107628477, 0.0013333558146428443, 0.00015403530393381608)
_LOG1P_C = (6.057847667939598e-09, 0.9999987830867273, -0.49995894468480306,
            0.3327853380006574, -0.24618967719166315, 0.18421386356488162,
            -0.12447194563436599, 0.06573552558543269, -0.0226280072114605,
            0.003662242215796141)


def _fast_exp(y):
    z0 = y * jnp.float32(1.4426950408889634)
    z = jnp.maximum(z0, -124.5)
    n = jnp.round(z)
    f = z - n
    p = jnp.float32(_EXP2_C[-1])
    for c in _EXP2_C[-2::-1]:
        p = p * f + jnp.float32(c)
    scale = lax.bitcast_convert_type(
        (n.astype(jnp.int32) + 127) << 23, jnp.float32)
    return jnp.where(z0 < -124.5, 0.0, p * scale)


def _fast_log1p01(v):
    p = jnp.float32(_LOG1P_C[-1])
    for c in _LOG1P_C[-2::-1]:
        p = p * v + jnp.float32(c)
    return p


def _tc_body(k_ref, w_ref, o_ref):
    a_max = jnp.max(k_ref[...])
    w_max = jnp.max(w_ref[...])
    x_hi = jnp.maximum(w_max, 0.0) + 0.7
    lim = jnp.minimum(a_max, jnp.float32(_MAX_A))

    def conv_cond(c):
        j, t = c
        return jnp.logical_and(j < lim, t > 1e-8)

    def conv_step(c):
        j, t = c
        return (j + 1.0, t * (x_hi / (j + 1.0)))

    trip_f, _ = lax.while_loop(conv_cond, conv_step,
                               (jnp.float32(0.0), jnp.float32(1.0)))
    trip = trip_f.astype(jnp.int32)

    n_sub = _TC_BLK // _SUB
    for i in range(n_sub):
        rows = pl.ds(i * _SUB, _SUB)
        w = w_ref[rows, :]
        a = k_ref[rows, :]
        v = _fast_exp(-jnp.abs(w))
        x = jnp.maximum(w, 0.0) + _fast_log1p01(v)
        e = _fast_exp(-x)

        def chunk(ci, carry):
            s, t = carry
            j0 = (ci * _CHUNK).astype(jnp.float32)
            for dj in range(_CHUNK):
                jf = j0 + jnp.float32(dj)
                s = s + jnp.where(a > jf, t, 0.0)
                t = t * (x * (1.0 / (jf + 1.0)))
            return (s, t)

        n_chunks = (trip + (_CHUNK - 1)) // _CHUNK
        s, _ = lax.fori_loop(0, n_chunks, chunk, (jnp.zeros_like(w), e))
        o_ref[rows, :] = 1.0 - s


def _tc_series(kin_exp, packed, p_rows):
    return pl.pallas_call(
        _tc_body,
        grid=(p_rows // _TC_BLK,),
        in_specs=[
            pl.BlockSpec((_TC_BLK, 128), lambda i: (i, 0)),
            pl.BlockSpec((_TC_BLK, 128), lambda i: (i, 0)),
        ],
        out_specs=pl.BlockSpec((_TC_BLK, 128), lambda i: (i, 0)),
        out_shape=jax.ShapeDtypeStruct((p_rows, 128), jnp.float32),
        compiler_params=pltpu.CompilerParams(
            dimension_semantics=("arbitrary",)),
    )(kin_exp, packed)


def kernel(problems, behavior_data, W):
    b, l = problems.shape
    dim = W.shape[1]
    n = b * l
    info = plsc.get_sparse_core_info()
    nw = info.num_cores * info.num_subcores

    idx3 = problems.reshape(nw, n // (nw * _GROUP), _GROUP)
    rows = _sc_gather(idx3, W, n, dim)

    p_rows = (n * dim) // 128
    packed = rows.reshape(p_rows, 128)
    a_flat = jnp.maximum(behavior_data.astype(jnp.float32) - 1.0, 0.0)
    kin_exp = jnp.repeat(a_flat.reshape(-1), dim).reshape(p_rows, 128)
    out = _tc_series(kin_exp, packed, p_rows)
    return out.reshape(b, l, dim)


# speedup vs baseline: 1.9621x; 1.3753x over previous
import functools

import jax
import jax.numpy as jnp
from jax import lax
from jax.experimental import pallas as pl
from jax.experimental.pallas import tpu as pltpu
from jax.experimental.pallas import tpu_sc as plsc

_GROUP = 128
_K = 5
_MAX_A = 48
_TC_BLK = 1024


def _sc_gather(idx3, table, n_rows, dim):
    info = plsc.get_sparse_core_info()
    nc, ns = info.num_cores, info.num_subcores
    nw = nc * ns
    rpw = n_rows // nw
    ng = rpw // _GROUP
    sup = ng // _K
    cg = _K * _GROUP

    @functools.partial(
        pl.kernel,
        out_type=jax.ShapeDtypeStruct((n_rows, dim), jnp.float32),
        mesh=plsc.VectorSubcoreMesh(core_axis_name="c", subcore_axis_name="s"),
        scratch_types=[
            pltpu.VMEM((ng, _GROUP), jnp.int32),
            pltpu.VMEM((cg, dim), jnp.float32),
            pltpu.VMEM((cg, dim), jnp.float32),
            pltpu.SemaphoreType.DMA,
            pltpu.SemaphoreType.DMA,
        ],
        compiler_params=pltpu.CompilerParams(use_tc_tiling_on_sc=False),
    )
    def gather_k(idx_hbm, table_hbm, out_hbm, idx_v, buf_a, buf_b, sem_a, sem_b):
        c = lax.axis_index("c")
        s = lax.axis_index("s")
        wid = s * nc + c
        base = wid * rpw
        pltpu.sync_copy(idx_hbm.at[wid], idx_v)

        def issue(sc_i, buf, sem):
            for j in range(_K):
                pltpu.async_copy(
                    table_hbm.at[idx_v.at[sc_i * _K + j]],
                    buf.at[pl.ds(j * _GROUP, _GROUP)], sem)

        def drain(sc_i, buf, sem):
            for j in range(_K):
                pltpu.make_async_copy(
                    table_hbm.at[idx_v.at[sc_i * _K + j]],
                    buf.at[pl.ds(j * _GROUP, _GROUP)], sem).wait()

        def write(sc_i, buf):
            pltpu.sync_copy(buf, out_hbm.at[pl.ds(base + sc_i * cg, cg)])

        issue(0, buf_a, sem_a)

        def body(p, carry):
            sa = 2 * p
            sb = 2 * p + 1
            issue(sb, buf_b, sem_b)
            drain(sa, buf_a, sem_a)
            write(sa, buf_a)

            @pl.when(sb + 1 < sup)
            def _():
                issue(sb + 1, buf_a, sem_a)

            drain(sb, buf_b, sem_b)
            write(sb, buf_b)
            return carry

        lax.fori_loop(0, sup // 2, body, 0)

    return gather_k(idx3, table)


_SUB = 64
_CHUNK = 8

_EXP2_C = (1.0, 0.6931471805599453, 0.2402265069591007, 0.05550410866482158,
           0.009618129107628477, 0.0013333558146428443, 0.00015403530393381608)
_LOG1P_C = (6.057847667939598e-09, 0.9999987830867273, -0.49995894468480306,
            0.3327853380006574, -0.24618967719166315, 0.18421386356488162,
            -0.12447194563436599, 0.06573552558543269, -0.0226280072114605,
            0.003662242215796141)


def _fast_exp(y):
    z0 = y * jnp.float32(1.4426950408889634)
    z = jnp.maximum(z0, -124.5)
    n = jnp.round(z)
    f = z - n
    p = jnp.float32(_EXP2_C[-1])
    for c in _EXP2_C[-2::-1]:
        p = p * f + jnp.float32(c)
    scale = lax.bitcast_convert_type(
        (n.astype(jnp.int32) + 127) << 23, jnp.float32)
    return jnp.where(z0 < -124.5, 0.0, p * scale)


def _fast_log1p01(v):
    p = jnp.float32(_LOG1P_C[-1])
    for c in _LOG1P_C[-2::-1]:
        p = p * v + jnp.float32(c)
    return p


def _tc_body(k_ref, w_ref, o_ref, acc_ref):
    a_max = jnp.max(k_ref[...])
    w_max = jnp.max(w_ref[...])
    x_hi = jnp.maximum(w_max, 0.0) + 0.7
    lim = jnp.minimum(a_max, jnp.float32(_MAX_A))

    def conv_cond(c):
        j, t = c
        return jnp.logical_and(j < lim, t > 1e-8)

    def conv_step(c):
        j, t = c
        return (j + 1.0, t * (x_hi / (j + 1.0)))

    trip_f, _ = lax.while_loop(conv_cond, conv_step,
                               (jnp.float32(0.0), jnp.float32(1.0)))
    trip = trip_f.astype(jnp.int32)

    n_sub = _TC_BLK // _SUB
    for i in range(n_sub):
        rows = pl.ds(i * _SUB, _SUB)
        w = w_ref[rows, :]
        a = k_ref[rows, :]
        v = _fast_exp(-jnp.abs(w))
        x = jnp.maximum(w, 0.0) + _fast_log1p01(v)
        e = _fast_exp(-x)

        def chunk(ci, carry):
            s, t = carry
            j0 = (ci * _CHUNK).astype(jnp.float32)
            for dj in range(_CHUNK):
                jf = j0 + jnp.float32(dj)
                s = s + jnp.where(a > jf, t, 0.0)
                t = t * (x * (1.0 / (jf + 1.0)))
            return (s, t)

        n_chunks = (trip + (_CHUNK - 1)) // _CHUNK
        s, _ = lax.fori_loop(0, n_chunks, chunk, (jnp.zeros_like(w), e))
        acc_ref[rows, :] = 1.0 - s

    o_ref[...] = acc_ref[...].T


def _tc_series(kin_exp, packed, nb, nld):
    return pl.pallas_call(
        _tc_body,
        grid=(nb // _TC_BLK, nld // 128),
        in_specs=[
            pl.BlockSpec((_TC_BLK, 128), lambda i, j: (i, j)),
            pl.BlockSpec((_TC_BLK, 128), lambda i, j: (i, j)),
        ],
        out_specs=pl.BlockSpec((128, _TC_BLK), lambda i, j: (j, i)),
        out_shape=jax.ShapeDtypeStruct((nld, nb), jnp.float32),
        scratch_shapes=[pltpu.VMEM((_TC_BLK, 128), jnp.float32)],
        compiler_params=pltpu.CompilerParams(
            dimension_semantics=("arbitrary", "arbitrary")),
    )(kin_exp, packed)


def kernel(problems, behavior_data, W):
    b, l = problems.shape
    dim = W.shape[1]
    n = b * l
    info = plsc.get_sparse_core_info()
    nw = info.num_cores * info.num_subcores

    idx3 = problems.reshape(nw, n // (nw * _GROUP), _GROUP)
    rows = _sc_gather(idx3, W, n, dim)

    packed = rows.reshape(b, l * dim)
    a_flat = jnp.maximum(behavior_data.astype(jnp.float32) - 1.0, 0.0)
    kin_exp = jnp.repeat(a_flat.reshape(-1), dim).reshape(b, l * dim)
    out_t = _tc_series(kin_exp, packed, b, l * dim)
    return out_t.reshape(l, dim, b).transpose(2, 0, 1)


# speedup vs baseline: 2.0782x; 1.0591x over previous
import functools

import jax
import jax.numpy as jnp
from jax import lax
from jax.experimental import pallas as pl
from jax.experimental.pallas import tpu as pltpu
from jax.experimental.pallas import tpu_sc as plsc

_GROUP = 128
_K = 5
_MAX_A = 48
_TC_BLK = 1024


def _sc_gather(idx3, table, n_rows, dim):
    info = plsc.get_sparse_core_info()
    nc, ns = info.num_cores, info.num_subcores
    nw = nc * ns
    rpw = n_rows // nw
    ng = rpw // _GROUP
    sup = ng // _K
    cg = _K * _GROUP

    @functools.partial(
        pl.kernel,
        out_type=jax.ShapeDtypeStruct((n_rows, dim), jnp.float32),
        mesh=plsc.VectorSubcoreMesh(core_axis_name="c", subcore_axis_name="s"),
        scratch_types=[
            pltpu.VMEM((ng, _GROUP), jnp.int32),
            pltpu.VMEM((cg, dim), jnp.float32),
            pltpu.VMEM((cg, dim), jnp.float32),
            pltpu.SemaphoreType.DMA,
            pltpu.SemaphoreType.DMA,
        ],
        compiler_params=pltpu.CompilerParams(use_tc_tiling_on_sc=False),
    )
    def gather_k(idx_hbm, table_hbm, out_hbm, idx_v, buf_a, buf_b, sem_a, sem_b):
        c = lax.axis_index("c")
        s = lax.axis_index("s")
        wid = s * nc + c
        base = wid * rpw
        pltpu.sync_copy(idx_hbm.at[wid], idx_v)

        def issue(sc_i, buf, sem):
            for j in range(_K):
                pltpu.async_copy(
                    table_hbm.at[idx_v.at[sc_i * _K + j]],
                    buf.at[pl.ds(j * _GROUP, _GROUP)], sem)

        def drain(sc_i, buf, sem):
            for j in range(_K):
                pltpu.make_async_copy(
                    table_hbm.at[idx_v.at[sc_i * _K + j]],
                    buf.at[pl.ds(j * _GROUP, _GROUP)], sem).wait()

        def write(sc_i, buf):
            pltpu.sync_copy(buf, out_hbm.at[pl.ds(base + sc_i * cg, cg)])

        issue(0, buf_a, sem_a)

        def body(p, carry):
            sa = 2 * p
            sb = 2 * p + 1
            issue(sb, buf_b, sem_b)
            drain(sa, buf_a, sem_a)
            write(sa, buf_a)

            @pl.when(sb + 1 < sup)
            def _():
                issue(sb + 1, buf_a, sem_a)

            drain(sb, buf_b, sem_b)
            write(sb, buf_b)
            return carry

        lax.fori_loop(0, sup // 2, body, 0)

    return gather_k(idx3, table)


_SUB = 64
_CHUNK = 16

_EXP2_C = (1.0, 0.6931471805599453, 0.2402265069591007, 0.05550410866482158,
           0.009618129107628477, 0.0013333558146428443, 0.00015403530393381608)
_LOG1P_C = (6.057847667939598e-09, 0.9999987830867273, -0.49995894468480306,
            0.3327853380006574, -0.24618967719166315, 0.18421386356488162,
            -0.12447194563436599, 0.06573552558543269, -0.0226280072114605,
            0.003662242215796141)


def _fast_exp(y):
    z0 = y * jnp.float32(1.4426950408889634)
    z = jnp.maximum(z0, -124.5)
    n = jnp.round(z)
    f = z - n
    p = jnp.float32(_EXP2_C[-1])
    for c in _EXP2_C[-2::-1]:
        p = p * f + jnp.float32(c)
    scale = lax.bitcast_convert_type(
        (n.astype(jnp.int32) + 127) << 23, jnp.float32)
    return jnp.where(z0 < -124.5, 0.0, p * scale)


def _fast_log1p01(v):
    p = jnp.float32(_LOG1P_C[-1])
    for c in _LOG1P_C[-2::-1]:
        p = p * v + jnp.float32(c)
    return p


def _tc_body(k_ref, w_ref, o_ref, acc_ref):
    a_max = jnp.max(k_ref[...])
    w_max = jnp.max(w_ref[...])
    x_hi = jnp.maximum(w_max, 0.0) + 0.7
    lim = jnp.minimum(a_max, jnp.float32(_MAX_A))

    def conv_cond(c):
        j, t = c
        return jnp.logical_and(j < lim, t > 1e-8)

    def conv_step(c):
        j, t = c
        return (j + 1.0, t * (x_hi / (j + 1.0)))

    trip_f, _ = lax.while_loop(conv_cond, conv_step,
                               (jnp.float32(0.0), jnp.float32(1.0)))
    trip = trip_f.astype(jnp.int32)

    n_sub = _TC_BLK // _SUB
    for i in range(n_sub):
        rows = pl.ds(i * _SUB, _SUB)
        w = w_ref[rows, :]
        a = k_ref[rows, :]
        v = _fast_exp(-jnp.abs(w))
        x = jnp.maximum(w, 0.0) + _fast_log1p01(v)
        e = _fast_exp(-x)

        def chunk(ci, carry):
            s, t = carry
            j0 = (ci * _CHUNK).astype(jnp.float32)
            for dj in range(_CHUNK):
                jf = j0 + jnp.float32(dj)
                s = s + jnp.where(a > jf, t, 0.0)
                t = t * (x * (1.0 / (jf + 1.0)))
            return (s, t)

        n_chunks = (trip + (_CHUNK - 1)) // _CHUNK
        s, _ = lax.fori_loop(0, n_chunks, chunk, (jnp.zeros_like(w), e))
        acc_ref[rows, :] = 1.0 - s

    o_ref[...] = acc_ref[...].T


def _tc_series(kin_exp, packed, nb, nld):
    return pl.pallas_call(
        _tc_body,
        grid=(nb // _TC_BLK, nld // 128),
        in_specs=[
            pl.BlockSpec((_TC_BLK, 128), lambda i, j: (i, j)),
            pl.BlockSpec((_TC_BLK, 128), lambda i, j: (i, j)),
        ],
        out_specs=pl.BlockSpec((128, _TC_BLK), lambda i, j: (j, i)),
        out_shape=jax.ShapeDtypeStruct((nld, nb), jnp.float32),
        scratch_shapes=[pltpu.VMEM((_TC_BLK, 128), jnp.float32)],
        compiler_params=pltpu.CompilerParams(
            dimension_semantics=("arbitrary", "arbitrary")),
    )(kin_exp, packed)


def kernel(problems, behavior_data, W):
    b, l = problems.shape
    dim = W.shape[1]
    n = b * l
    info = plsc.get_sparse_core_info()
    nw = info.num_cores * info.num_subcores

    idx3 = problems.reshape(nw, n // (nw * _GROUP), _GROUP)
    rows = _sc_gather(idx3, W, n, dim)

    packed = rows.reshape(b, l * dim)
    a_flat = jnp.maximum(behavior_data.astype(jnp.float32) - 1.0, 0.0)
    kin_exp = jnp.repeat(a_flat.reshape(-1), dim).reshape(b, l * dim)
    out_t = _tc_series(kin_exp, packed, b, l * dim)
    return out_t.reshape(l, dim, b).transpose(2, 0, 1)
